# baseline (device time: 39033 ns/iter reference)
import jax
import jax.numpy as jnp
from jax import lax
from jax.experimental import pallas as pl
from jax.experimental.pallas import tpu as pltpu


def kernel(Q, K, V):
    b, q_len, h, d = Q.shape
    k_len = K.shape[1]
    scale = d ** -0.5

    def body(q_ref, k_ref, v_ref, o_ref, comm_ref, stats_ref, send_sems, recv_sems):
        my_x = lax.axis_index("x")
        my_y = lax.axis_index("y")
        my_z = lax.axis_index("z")
        peer = (my_x, 1 - my_y, my_z)

        q = q_ref[:, 0]
        k = k_ref[...]
        v = v_ref[...]
        s = jnp.sum(q[:, None] * k, axis=-1) * scale
        m = jnp.max(s, axis=1)
        p = jnp.exp(s - m[:, None, :])
        l = jnp.sum(p, axis=1)
        o = jnp.sum(p[..., None] * v, axis=1)

        comm_ref[0] = o
        stats_ref[0, 0] = m
        stats_ref[0, 1] = l

        barrier_sem = pltpu.get_barrier_semaphore()
        pl.semaphore_signal(
            barrier_sem, inc=1, device_id=peer,
            device_id_type=pl.DeviceIdType.MESH,
        )
        pl.semaphore_wait(barrier_sem, 1)

        rdma_o = pltpu.make_async_remote_copy(
            src_ref=comm_ref.at[0], dst_ref=comm_ref.at[1],
            send_sem=send_sems.at[0], recv_sem=recv_sems.at[0],
            device_id=peer, device_id_type=pl.DeviceIdType.MESH,
        )
        rdma_s = pltpu.make_async_remote_copy(
            src_ref=stats_ref.at[0], dst_ref=stats_ref.at[1],
            send_sem=send_sems.at[1], recv_sem=recv_sems.at[1],
            device_id=peer, device_id_type=pl.DeviceIdType.MESH,
        )
        rdma_o.start()
        rdma_s.start()
        rdma_o.wait()
        rdma_s.wait()

        m_pe = stats_ref[1, 0]
        l_pe = stats_ref[1, 1]
        o_pe = comm_ref[1]

        m_new = jnp.maximum(m, m_pe)
        a_me = jnp.exp(m - m_new)
        a_pe = jnp.exp(m_pe - m_new)
        l_new = l * a_me + l_pe * a_pe
        o_new = (o * a_me[..., None] + o_pe * a_pe[..., None]) / l_new[..., None]
        o_ref[:, 0] = o_new

    return pl.pallas_call(
        body,
        out_shape=jax.ShapeDtypeStruct((b, q_len, h, d), jnp.float32),
        in_specs=[
            pl.BlockSpec(memory_space=pltpu.VMEM),
            pl.BlockSpec(memory_space=pltpu.VMEM),
            pl.BlockSpec(memory_space=pltpu.VMEM),
        ],
        out_specs=pl.BlockSpec(memory_space=pltpu.VMEM),
        scratch_shapes=[
            pltpu.VMEM((2, b, h, d), jnp.float32),
            pltpu.VMEM((2, 2, b, h), jnp.float32),
            pltpu.SemaphoreType.DMA((2,)),
            pltpu.SemaphoreType.DMA((2,)),
        ],
        compiler_params=pltpu.CompilerParams(collective_id=0),
    )(Q, K, V)


# device time: 20129 ns/iter; 1.9391x vs baseline; 1.9391x over previous
import jax
import jax.numpy as jnp
from jax import lax
from jax.experimental import pallas as pl
from jax.experimental.pallas import tpu as pltpu


def kernel(Q, K, V):
    b, q_len, h, d = Q.shape
    k_len = K.shape[1]
    hd = h * d
    scale = d ** -0.5
    K2 = K.reshape(b, k_len, hd)
    V2 = V.reshape(b, k_len, hd)

    def body(q_ref, k_ref, v_ref, o_ref, comm_ref, stats_ref, send_sems, recv_sems):
        my_x = lax.axis_index("x")
        my_y = lax.axis_index("y")
        my_z = lax.axis_index("z")
        peer = (my_x, 1 - my_y, my_z)

        hp = lax.broadcasted_iota(jnp.int32, (h, hd), 1) // d
        hh = lax.broadcasted_iota(jnp.int32, (h, hd), 0)
        mask = (hp == hh).astype(jnp.float32)

        for bi in range(b):
            qb = q_ref[bi, 0]
            qbd = (jnp.concatenate([qb] * h, axis=1) * mask).astype(jnp.bfloat16)
            kb = k_ref[bi].astype(jnp.bfloat16)
            s = jax.lax.dot_general(
                qbd, kb, (((1,), (1,)), ((), ())),
                preferred_element_type=jnp.float32) * scale
            m = jnp.max(s, axis=1, keepdims=True)
            p = jnp.exp(s - m)
            l = jnp.sum(p, axis=1, keepdims=True)
            vb = v_ref[bi].astype(jnp.bfloat16)
            r = jax.lax.dot_general(
                p.astype(jnp.bfloat16), vb, (((1,), (0,)), ((), ())),
                preferred_element_type=jnp.float32)
            rm = r * mask
            o = rm[:, 0:d]
            for hi in range(1, h):
                o = o + rm[:, hi * d:(hi + 1) * d]
            comm_ref[0, bi] = o
            stats_ref[0, bi] = jnp.concatenate([m, l], axis=1)

        barrier_sem = pltpu.get_barrier_semaphore()
        pl.semaphore_signal(
            barrier_sem, inc=1, device_id=peer,
            device_id_type=pl.DeviceIdType.MESH,
        )
        pl.semaphore_wait(barrier_sem, 1)

        rdma_o = pltpu.make_async_remote_copy(
            src_ref=comm_ref.at[0], dst_ref=comm_ref.at[1],
            send_sem=send_sems.at[0], recv_sem=recv_sems.at[0],
            device_id=peer, device_id_type=pl.DeviceIdType.MESH,
        )
        rdma_s = pltpu.make_async_remote_copy(
            src_ref=stats_ref.at[0], dst_ref=stats_ref.at[1],
            send_sem=send_sems.at[1], recv_sem=recv_sems.at[1],
            device_id=peer, device_id_type=pl.DeviceIdType.MESH,
        )
        rdma_o.start()
        rdma_s.start()
        rdma_o.wait()
        rdma_s.wait()

        m_me = stats_ref[0, :, :, 0]
        l_me = stats_ref[0, :, :, 1]
        o_me = comm_ref[0]
        m_pe = stats_ref[1, :, :, 0]
        l_pe = stats_ref[1, :, :, 1]
        o_pe = comm_ref[1]

        m_new = jnp.maximum(m_me, m_pe)
        a_me = jnp.exp(m_me - m_new)
        a_pe = jnp.exp(m_pe - m_new)
        l_new = l_me * a_me + l_pe * a_pe
        o_new = (o_me * a_me[..., None] + o_pe * a_pe[..., None]) / l_new[..., None]
        o_ref[:, 0] = o_new

    return pl.pallas_call(
        body,
        out_shape=jax.ShapeDtypeStruct((b, q_len, h, d), jnp.float32),
        in_specs=[
            pl.BlockSpec(memory_space=pltpu.VMEM),
            pl.BlockSpec(memory_space=pltpu.VMEM),
            pl.BlockSpec(memory_space=pltpu.VMEM),
        ],
        out_specs=pl.BlockSpec(memory_space=pltpu.VMEM),
        scratch_shapes=[
            pltpu.VMEM((2, b, h, d), jnp.float32),
            pltpu.VMEM((2, b, h, 2), jnp.float32),
            pltpu.SemaphoreType.DMA((2,)),
            pltpu.SemaphoreType.DMA((2,)),
        ],
        compiler_params=pltpu.CompilerParams(collective_id=0),
    )(Q, K2, V2)
